# baseline (speedup 1.0000x reference)
import functools

import jax
import jax.numpy as jnp
from jax import lax
from jax.experimental import pallas as pl
from jax.experimental.pallas import tpu as pltpu
from jax.experimental.pallas import tpu_sc as plsc

VOCAB = 1000000
EMB = 64
NUM_CLASSES = 1000
B = 4096
L = 200

CHUNK = 128
NBUF = 8

NC = 2
NS = 16
NW = NC * NS
B_PER_W = B // NW
IDX_PER_W = B_PER_W * L
CHUNKS = IDX_PER_W // CHUNK


def _sc_pool(x3, seg, table):
    mesh = plsc.VectorSubcoreMesh(core_axis_name="c", subcore_axis_name="s")

    @functools.partial(
        pl.kernel,
        out_type=jax.ShapeDtypeStruct((B, EMB), jnp.float32),
        mesh=mesh,
        compiler_params=pltpu.CompilerParams(use_tc_tiling_on_sc=False),
        scratch_types=[
            pltpu.VMEM((CHUNKS + NBUF, CHUNK), jnp.int32),
            pltpu.VMEM((CHUNKS, CHUNK), jnp.int32),
        ] + [pltpu.VMEM((CHUNK, EMB), jnp.float32)] * NBUF + [
            pltpu.VMEM_SHARED((NS * B_PER_W, EMB), jnp.float32),
        ] + [pltpu.SemaphoreType.DMA] * NBUF,
    )
    def pool(x_hbm, seg_hbm, table_hbm, out_hbm, idx_v, dst_v, *rest):
        bufs = rest[:NBUF]
        acc_sh = rest[NBUF]
        sems = rest[NBUF + 1:]
        buf0 = bufs[0]
        s = lax.axis_index("s")
        wid = s * NC + lax.axis_index("c")
        base = wid * B_PER_W

        pltpu.sync_copy(x_hbm.at[wid], idx_v.at[pl.ds(0, CHUNKS)])
        pltpu.sync_copy(seg_hbm, dst_v)

        zeros_i = jnp.zeros((16,), jnp.int32)

        @pl.loop(CHUNKS, CHUNKS + NBUF)
        def _(k):
            for j in range(CHUNK // 16):
                idx_v[k, pl.ds(j * 16, 16)] = zeros_i

        sbase = jnp.full((16,), s * B_PER_W, jnp.int32)

        @pl.loop(0, CHUNKS)
        def _(k):
            for j in range(CHUNK // 16):
                sl = pl.ds(j * 16, 16)
                dst_v[k, sl] = dst_v[k, sl] + sbase

        zeros_f = jnp.zeros((16,), jnp.float32)

        @pl.loop(0, CHUNK)
        def _(r):
            for j in range(EMB // 16):
                buf0[r, pl.ds(j * 16, 16)] = zeros_f

        pltpu.sync_copy(buf0, acc_sh.at[pl.ds(s * B_PER_W, B_PER_W)])

        for j in range(NBUF):
            pltpu.async_copy(table_hbm.at[idx_v.at[j]], bufs[j], sems[j])

        @pl.loop(0, CHUNKS, step=NBUF)
        def _(k):
            for j in range(NBUF):
                pltpu.make_async_copy(table_hbm.at[idx_v.at[k + j]],
                                      bufs[j], sems[j]).wait()
                pltpu.sync_copy(bufs[j], acc_sh.at[dst_v.at[k + j]],
                                add=True)
                pltpu.async_copy(table_hbm.at[idx_v.at[k + NBUF + j]],
                                 bufs[j], sems[j])

        for j in range(NBUF):
            pltpu.make_async_copy(table_hbm.at[idx_v.at[j]],
                                  bufs[j], sems[j]).wait()

        pltpu.sync_copy(acc_sh.at[pl.ds(s * B_PER_W, B_PER_W)],
                        out_hbm.at[pl.ds(base, B_PER_W)])

    return pool(x3, seg, table)


def _tc_head(sums, fc_wt, fc_b2):
    TB = 256

    def body(s_ref, w_ref, b_ref, o_ref):
        o_ref[...] = (
            jnp.dot(s_ref[...], w_ref[...],
                    preferred_element_type=jnp.float32,
                    precision=lax.Precision.HIGHEST) * (1.0 / L)
            + b_ref[...]
        )

    return pl.pallas_call(
        body,
        grid=(B // TB,),
        in_specs=[
            pl.BlockSpec((TB, EMB), lambda i: (i, 0)),
            pl.BlockSpec((EMB, NUM_CLASSES), lambda i: (0, 0)),
            pl.BlockSpec((1, NUM_CLASSES), lambda i: (0, 0)),
        ],
        out_specs=pl.BlockSpec((TB, NUM_CLASSES), lambda i: (i, 0)),
        out_shape=jax.ShapeDtypeStruct((B, NUM_CLASSES), jnp.float32),
    )(sums, fc_wt, fc_b2)


def kernel(x, table, fc_w, fc_b):
    x3 = x.astype(jnp.int32).reshape(NW, CHUNKS, CHUNK)
    seg = (jnp.arange(CHUNKS * CHUNK, dtype=jnp.int32) // L).reshape(CHUNKS, CHUNK)
    sums = _sc_pool(x3, seg, table)
    return _tc_head(sums, fc_w.T, fc_b.reshape(1, NUM_CLASSES))


# speedup vs baseline: 1.6943x; 1.6943x over previous
import functools

import jax
import jax.numpy as jnp
from jax import lax
from jax.experimental import pallas as pl
from jax.experimental.pallas import tpu as pltpu
from jax.experimental.pallas import tpu_sc as plsc

VOCAB = 1000000
EMB = 64
NUM_CLASSES = 1000
B = 4096
L = 200

CHUNK = 128
NBUF = 8

NC = 2
NS = 16
NW = NC * NS
B_PER_W = B // NW
IDX_PER_W = B_PER_W * L
CHUNKS = IDX_PER_W // CHUNK


def _sc_pool(x3, seg, table):
    mesh = plsc.VectorSubcoreMesh(core_axis_name="c", subcore_axis_name="s")

    @functools.partial(
        pl.kernel,
        out_type=jax.ShapeDtypeStruct((B, EMB), jnp.float32),
        mesh=mesh,
        compiler_params=pltpu.CompilerParams(use_tc_tiling_on_sc=False),
        scratch_types=[
            pltpu.VMEM((CHUNKS, CHUNK), jnp.int32),
            pltpu.VMEM((CHUNKS, CHUNK), jnp.int32),
        ] + [pltpu.VMEM((CHUNK, EMB), jnp.float32)] * NBUF + [
            pltpu.VMEM_SHARED((NS * B_PER_W, EMB), jnp.float32),
        ] + [pltpu.SemaphoreType.DMA] * NBUF,
    )
    def pool(x_hbm, seg_hbm, table_hbm, out_hbm, idx_v, dst_v, *rest):
        bufs = rest[:NBUF]
        acc_sh = rest[NBUF]
        sems = rest[NBUF + 1:]
        buf0 = bufs[0]
        s = lax.axis_index("s")
        wid = s * NC + lax.axis_index("c")
        base = wid * B_PER_W

        pltpu.sync_copy(x_hbm.at[wid], idx_v)
        pltpu.sync_copy(seg_hbm, dst_v)

        sbase = jnp.full((16,), s * B_PER_W, jnp.int32)

        @pl.loop(0, CHUNKS)
        def _(k):
            for j in range(CHUNK // 16):
                sl = pl.ds(j * 16, 16)
                dst_v[k, sl] = dst_v[k, sl] + sbase

        zeros_f = jnp.zeros((16,), jnp.float32)

        @pl.loop(0, CHUNK)
        def _(r):
            for j in range(EMB // 16):
                buf0[r, pl.ds(j * 16, 16)] = zeros_f

        pltpu.sync_copy(buf0, acc_sh.at[pl.ds(s * B_PER_W, B_PER_W)])

        @pl.loop(0, CHUNKS, step=NBUF)
        def _(k):
            cps = [
                pltpu.async_copy(table_hbm.at[idx_v.at[k + j]],
                                 bufs[j], sems[j])
                for j in range(NBUF)
            ]
            for j in range(NBUF):
                cps[j].wait()
                pltpu.sync_copy(bufs[j], acc_sh.at[dst_v.at[k + j]],
                                add=True)

        pltpu.sync_copy(acc_sh.at[pl.ds(s * B_PER_W, B_PER_W)],
                        out_hbm.at[pl.ds(base, B_PER_W)])

    return pool(x3, seg, table)


def _tc_head(sums, fc_wt, fc_b2):
    TB = 256

    def body(s_ref, w_ref, b_ref, o_ref):
        o_ref[...] = (
            jnp.dot(s_ref[...], w_ref[...],
                    preferred_element_type=jnp.float32,
                    precision=lax.Precision.HIGHEST) * (1.0 / L)
            + b_ref[...]
        )

    return pl.pallas_call(
        body,
        grid=(B // TB,),
        in_specs=[
            pl.BlockSpec((TB, EMB), lambda i: (i, 0)),
            pl.BlockSpec((EMB, NUM_CLASSES), lambda i: (0, 0)),
            pl.BlockSpec((1, NUM_CLASSES), lambda i: (0, 0)),
        ],
        out_specs=pl.BlockSpec((TB, NUM_CLASSES), lambda i: (i, 0)),
        out_shape=jax.ShapeDtypeStruct((B, NUM_CLASSES), jnp.float32),
    )(sums, fc_wt, fc_b2)


def kernel(x, table, fc_w, fc_b):
    x3 = x.astype(jnp.int32).reshape(NW, CHUNKS, CHUNK)
    seg = (jnp.arange(CHUNKS * CHUNK, dtype=jnp.int32) // L).reshape(CHUNKS, CHUNK)
    sums = _sc_pool(x3, seg, table)
    return _tc_head(sums, fc_w.T, fc_b.reshape(1, NUM_CLASSES))
